# baseline (speedup 1.0000x reference)
import functools

import jax
import jax.numpy as jnp
from jax import lax
from jax.experimental import pallas as pl
from jax.experimental.pallas import tpu as pltpu
from jax.experimental.pallas import tpu_sc as plsc

B, F, V, D = 4096, 26, 100000, 16
NC, NS = 2, 16
NW = NC * NS
BPW = B // NW
NCHUNK = BPW // 16

_mesh = plsc.VectorSubcoreMesh(core_axis_name="c", subcore_axis_name="s")


@functools.partial(
    pl.kernel,
    mesh=_mesh,
    compiler_params=pltpu.CompilerParams(
        needs_layout_passes=False, use_tc_tiling_on_sc=False),
    out_type=jax.ShapeDtypeStruct((B,), jnp.float32),
    scratch_types=[
        pltpu.VMEM((F, BPW), jnp.int32),
        pltpu.VMEM((F, BPW), jnp.int32),
        pltpu.VMEM((F * BPW, D), jnp.float32),
        pltpu.VMEM((F, BPW), jnp.float32),
        pltpu.VMEM((D, BPW), jnp.float32),
        pltpu.VMEM((BPW,), jnp.float32),
        pltpu.VMEM((16,), jnp.float32),
        pltpu.SemaphoreType.DMA,
        pltpu.SemaphoreType.DMA,
    ],
)
def _fm_sc(idx_hbm, tab_hbm, w_hbm, bias_hbm, out_hbm,
           idx_v, idxo_v, emb_v, w_v, p_v, out_v, bias_v, sem_e, sem_w):
    wid = lax.axis_index("s") * NC + lax.axis_index("c")
    pltpu.sync_copy(idx_hbm.at[:, pl.ds(wid * BPW, BPW)], idx_v)

    lanes = lax.iota(jnp.int32, 16)
    zeros = jnp.zeros((16,), jnp.float32)

    copies = []
    for f in range(F):
        for s in range(NCHUNK):
            idxo_v[f, pl.ds(s * 16, 16)] = idx_v[f, pl.ds(s * 16, 16)] + f * V
        copies.append(pltpu.async_copy(
            tab_hbm.at[idxo_v.at[f]], emb_v.at[pl.ds(f * BPW, BPW)], sem_e))
        copies.append(pltpu.async_copy(
            w_hbm.at[f].at[idx_v.at[f]], w_v.at[f], sem_w))
    pltpu.sync_copy(bias_hbm, bias_v)
    for cp in copies:
        cp.wait()

    def body(i, c):
        acc = zeros
        acc2 = zeros
        for f in range(F):
            v = emb_v[f * BPW + i, :]
            acc = acc + v
            acc2 = acc2 + v * v
        p = acc * acc - acc2
        plsc.store_scatter(p_v, [lanes, jnp.full((16,), i, jnp.int32)], p)
        return c

    lax.fori_loop(0, BPW, body, 0)

    bias_vec = bias_v[...]
    for s in range(NCHUNK):
        lin = bias_vec
        for f in range(F):
            lin = lin + w_v[f, pl.ds(s * 16, 16)]
        inter = zeros
        for d in range(D):
            inter = inter + p_v[d, pl.ds(s * 16, 16)]
        x = lin + 0.5 * inter
        out_v[pl.ds(s * 16, 16)] = 1.0 / (1.0 + jnp.exp(-x))

    pltpu.sync_copy(out_v, out_hbm.at[pl.ds(wid * BPW, BPW)])


def kernel(indices, tables, w_linear, bias):
    idx_t = indices.astype(jnp.int32).T
    tab = tables.reshape(F * V, D)
    bias_vec = jnp.broadcast_to(bias.astype(jnp.float32), (16,))
    return _fm_sc(idx_t, tab, w_linear, bias_vec)


# speedup vs baseline: 3.0287x; 3.0287x over previous
import functools

import jax
import jax.numpy as jnp
from jax import lax
from jax.experimental import pallas as pl
from jax.experimental.pallas import tpu as pltpu
from jax.experimental.pallas import tpu_sc as plsc

B, F, V, D = 4096, 26, 100000, 16
NC, NS = 2, 16
NW = NC * NS
BPW = B // NW
NCHUNK = BPW // 16

_mesh = plsc.VectorSubcoreMesh(core_axis_name="c", subcore_axis_name="s")


@functools.partial(
    pl.kernel,
    mesh=_mesh,
    compiler_params=pltpu.CompilerParams(
        needs_layout_passes=False, use_tc_tiling_on_sc=False),
    out_type=jax.ShapeDtypeStruct((B,), jnp.float32),
    scratch_types=[
        pltpu.VMEM((F, BPW), jnp.int32),
        pltpu.VMEM((F * D, BPW), jnp.float32),
        pltpu.VMEM((F, BPW), jnp.float32),
        pltpu.VMEM((BPW,), jnp.float32),
        pltpu.VMEM((16,), jnp.float32),
        pltpu.SemaphoreType.DMA,
        pltpu.SemaphoreType.DMA,
    ],
)
def _fm_sc(idx_hbm, tab_hbm, w_hbm, bias_hbm, out_hbm,
           idx_v, emb_v, w_v, out_v, bias_v, sem_e, sem_w):
    wid = lax.axis_index("s") * NC + lax.axis_index("c")
    pltpu.sync_copy(idx_hbm.at[:, pl.ds(wid * BPW, BPW)], idx_v)

    zeros = jnp.zeros((16,), jnp.float32)

    window = []
    for f in range(F):
        row = idx_v.at[f]
        fired = [pltpu.async_copy(
            tab_hbm.at[f, d].at[row], emb_v.at[f * D + d], sem_e)
            for d in range(D)]
        fired.append(pltpu.async_copy(w_hbm.at[f].at[row], w_v.at[f], sem_w))
        window.append(fired)
        if len(window) > 2:
            for cp in window.pop(0):
                cp.wait()
    pltpu.sync_copy(bias_hbm, bias_v)
    for fired in window:
        for cp in fired:
            cp.wait()

    bias_vec = bias_v[...]
    for s in range(NCHUNK):
        def dbody(d, inter, s=s):
            acc = zeros
            acc2 = zeros
            for f in range(F):
                v = emb_v[f * D + d, pl.ds(s * 16, 16)]
                acc = acc + v
                acc2 = acc2 + v * v
            return inter + (acc * acc - acc2)

        inter = lax.fori_loop(0, D, dbody, zeros)
        lin = bias_vec
        for f in range(F):
            lin = lin + w_v[f, pl.ds(s * 16, 16)]
        x = lin + 0.5 * inter
        out_v[pl.ds(s * 16, 16)] = 1.0 / (1.0 + jnp.exp(-x))

    pltpu.sync_copy(out_v, out_hbm.at[pl.ds(wid * BPW, BPW)])


def kernel(indices, tables, w_linear, bias):
    idx_t = indices.astype(jnp.int32).T
    tab_t = tables.transpose(0, 2, 1)
    bias_vec = jnp.broadcast_to(bias.astype(jnp.float32), (16,))
    return _fm_sc(idx_t, tab_t, w_linear, bias_vec)


# speedup vs baseline: 3.0923x; 1.0210x over previous
import functools

import jax
import jax.numpy as jnp
from jax import lax
from jax.experimental import pallas as pl
from jax.experimental.pallas import tpu as pltpu
from jax.experimental.pallas import tpu_sc as plsc

B, F, V, D = 4096, 26, 100000, 16
NC, NS = 2, 16
NW = NC * NS
BPW = B // NW
NCHUNK = BPW // 16

_mesh = plsc.VectorSubcoreMesh(core_axis_name="c", subcore_axis_name="s")


@functools.partial(
    pl.kernel,
    mesh=_mesh,
    compiler_params=pltpu.CompilerParams(
        needs_layout_passes=False, use_tc_tiling_on_sc=False),
    out_type=jax.ShapeDtypeStruct((B,), jnp.float32),
    scratch_types=[
        pltpu.VMEM((F, BPW), jnp.int32),
        pltpu.VMEM((F * D, BPW), jnp.float32),
        pltpu.VMEM((F, BPW), jnp.float32),
        pltpu.VMEM((D, BPW), jnp.float32),
        pltpu.VMEM((D, BPW), jnp.float32),
        pltpu.VMEM((BPW,), jnp.float32),
        pltpu.VMEM((16,), jnp.float32),
        pltpu.SemaphoreType.DMA,
        pltpu.SemaphoreType.DMA,
    ],
)
def _fm_sc(idx_hbm, tab_hbm, w_hbm, bias_hbm, out_hbm,
           idx_v, emb_v, w_v, acc_v, acc2_v, out_v, bias_v, sem_e, sem_w):
    wid = lax.axis_index("s") * NC + lax.axis_index("c")
    pltpu.sync_copy(idx_hbm.at[:, pl.ds(wid * BPW, BPW)], idx_v)

    zeros = jnp.zeros((16,), jnp.float32)

    def accumulate(f, init):
        def dbody(d, c, f=f, init=init):
            for s in range(NCHUNK):
                v = emb_v[f * D + d, pl.ds(s * 16, 16)]
                if init:
                    acc_v[d, pl.ds(s * 16, 16)] = v
                    acc2_v[d, pl.ds(s * 16, 16)] = v * v
                else:
                    acc_v[d, pl.ds(s * 16, 16)] = (
                        acc_v[d, pl.ds(s * 16, 16)] + v)
                    acc2_v[d, pl.ds(s * 16, 16)] = (
                        acc2_v[d, pl.ds(s * 16, 16)] + v * v)
            return c
        lax.fori_loop(0, D, dbody, 0)

    window = []
    done = 0
    for f in range(F):
        row = idx_v.at[f]
        fired = [pltpu.async_copy(
            tab_hbm.at[f, d].at[row], emb_v.at[f * D + d], sem_e)
            for d in range(D)]
        fired.append(pltpu.async_copy(w_hbm.at[f].at[row], w_v.at[f], sem_w))
        window.append(fired)
        if len(window) > 2:
            for cp in window.pop(0):
                cp.wait()
            accumulate(done, done == 0)
            done += 1
    pltpu.sync_copy(bias_hbm, bias_v)
    for fired in window:
        for cp in fired:
            cp.wait()
        accumulate(done, done == 0)
        done += 1

    bias_vec = bias_v[...]
    for s in range(NCHUNK):
        inter = zeros
        for d in range(D):
            a = acc_v[d, pl.ds(s * 16, 16)]
            inter = inter + (a * a - acc2_v[d, pl.ds(s * 16, 16)])
        lin = bias_vec
        for f in range(F):
            lin = lin + w_v[f, pl.ds(s * 16, 16)]
        x = lin + 0.5 * inter
        out_v[pl.ds(s * 16, 16)] = 1.0 / (1.0 + jnp.exp(-x))

    pltpu.sync_copy(out_v, out_hbm.at[pl.ds(wid * BPW, BPW)])


def kernel(indices, tables, w_linear, bias):
    idx_t = indices.astype(jnp.int32).T
    tab_t = tables.transpose(0, 2, 1)
    bias_vec = jnp.broadcast_to(bias.astype(jnp.float32), (16,))
    return _fm_sc(idx_t, tab_t, w_linear, bias_vec)


# speedup vs baseline: 3.1068x; 1.0047x over previous
import functools

import jax
import jax.numpy as jnp
from jax import lax
from jax.experimental import pallas as pl
from jax.experimental.pallas import tpu as pltpu
from jax.experimental.pallas import tpu_sc as plsc

B, F, V, D = 4096, 26, 100000, 16
NC, NS = 2, 16
NW = NC * NS
BPW = B // NW
NCHUNK = BPW // 16

_mesh = plsc.VectorSubcoreMesh(core_axis_name="c", subcore_axis_name="s")


@functools.partial(
    pl.kernel,
    mesh=_mesh,
    compiler_params=pltpu.CompilerParams(
        needs_layout_passes=False, use_tc_tiling_on_sc=False),
    out_type=jax.ShapeDtypeStruct((B,), jnp.float32),
    scratch_types=[
        pltpu.VMEM((F, BPW), jnp.int32),
        pltpu.VMEM((F * D, BPW), jnp.float32),
        pltpu.VMEM((F, BPW), jnp.float32),
        pltpu.VMEM((D, BPW), jnp.float32),
        pltpu.VMEM((D, BPW), jnp.float32),
        pltpu.VMEM((BPW,), jnp.float32),
        pltpu.VMEM((16,), jnp.float32),
        pltpu.SemaphoreType.DMA,
        pltpu.SemaphoreType.DMA,
    ],
)
def _fm_sc(idx_hbm, tab_hbm, w_hbm, bias_hbm, out_hbm,
           idx_v, emb_v, w_v, acc_v, acc2_v, out_v, bias_v, sem_e, sem_w):
    wid = lax.axis_index("s") * NC + lax.axis_index("c")
    pltpu.sync_copy(idx_hbm.at[:, pl.ds(wid * BPW, BPW)], idx_v)

    zeros = jnp.zeros((16,), jnp.float32)

    def accumulate(f, init):
        def dbody(d, c, f=f, init=init):
            for s in range(NCHUNK):
                v = emb_v[f * D + d, pl.ds(s * 16, 16)]
                if init:
                    acc_v[d, pl.ds(s * 16, 16)] = v
                    acc2_v[d, pl.ds(s * 16, 16)] = v * v
                else:
                    acc_v[d, pl.ds(s * 16, 16)] = (
                        acc_v[d, pl.ds(s * 16, 16)] + v)
                    acc2_v[d, pl.ds(s * 16, 16)] = (
                        acc2_v[d, pl.ds(s * 16, 16)] + v * v)
            return c
        lax.fori_loop(0, D, dbody, 0)

    window = []
    done = 0
    for f in range(F):
        row = idx_v.at[f]
        fired = [pltpu.async_copy(
            tab_hbm.at[f, d].at[row], emb_v.at[f * D + d], sem_e)
            for d in range(D)]
        fired.append(pltpu.async_copy(w_hbm.at[f].at[row], w_v.at[f], sem_w))
        window.append(fired)
        if len(window) > 3:
            for cp in window.pop(0):
                cp.wait()
            accumulate(done, done == 0)
            done += 1
    pltpu.sync_copy(bias_hbm, bias_v)
    for fired in window:
        for cp in fired:
            cp.wait()
        accumulate(done, done == 0)
        done += 1

    bias_vec = bias_v[...]
    for s in range(NCHUNK):
        inter = zeros
        for d in range(D):
            a = acc_v[d, pl.ds(s * 16, 16)]
            inter = inter + (a * a - acc2_v[d, pl.ds(s * 16, 16)])
        lin = bias_vec
        for f in range(F):
            lin = lin + w_v[f, pl.ds(s * 16, 16)]
        x = lin + 0.5 * inter
        out_v[pl.ds(s * 16, 16)] = 1.0 / (1.0 + jnp.exp(-x))

    pltpu.sync_copy(out_v, out_hbm.at[pl.ds(wid * BPW, BPW)])


def kernel(indices, tables, w_linear, bias):
    idx_t = indices.astype(jnp.int32).T
    tab_t = tables.transpose(0, 2, 1)
    bias_vec = jnp.broadcast_to(bias.astype(jnp.float32), (16,))
    return _fm_sc(idx_t, tab_t, w_linear, bias_vec)
